# baseline (speedup 1.0000x reference)
import jax
import jax.numpy as jnp
from jax.experimental import pallas as pl


def _gcn2_kernel(xp_ref, adj_ref, w1_ref, b1_ref, w2_ref, b2_ref, out_ref):
    adjv = adj_ref[...]
    n = adjv.shape[0]
    dis = jax.lax.rsqrt(1.0 + jnp.sum(adjv, axis=0, keepdims=True))
    r = jax.lax.broadcasted_iota(jnp.int32, (n, n), 0)
    c = jax.lax.broadcasted_iota(jnp.int32, (n, n), 1)
    ahat = (adjv + jnp.where(r == c, 1.0, 0.0)) * dis

    w1t = w1_ref[...].T
    w2t = w2_ref[...].T
    b1 = b1_ref[...]
    b2 = b2_ref[...]
    f1 = w1t.shape[1]
    f2 = w2t.shape[1]
    nb = xp_ref.shape[0] // f1

    def layer(hcur, wt, f, bias):
        agg = jnp.dot(hcur * dis, ahat, preferred_element_type=jnp.float32)
        pieces = [
            jnp.maximum(
                jnp.dot(wt, agg[i * f:(i + 1) * f, :],
                        preferred_element_type=jnp.float32) + bias, 0.0)
            for i in range(nb)
        ]
        return jnp.concatenate(pieces, axis=0)

    h1 = layer(xp_ref[...], w1t, f1, b1)
    out_ref[...] = layer(h1, w2t, f2, b2)


def kernel(x, adj, W1, b1, W2, b2):
    bt, n, f = x.shape
    h = W1.shape[1]
    o = W2.shape[1]
    xp = x.transpose(0, 2, 1).reshape(bt * f, n)
    grid = 4
    rows_in = bt * f // grid
    rows_out = bt * o // grid
    outp = pl.pallas_call(
        _gcn2_kernel,
        grid=(grid,),
        in_specs=[
            pl.BlockSpec((rows_in, n), lambda i: (i, 0)),
            pl.BlockSpec((n, n), lambda i: (0, 0)),
            pl.BlockSpec((f, h), lambda i: (0, 0)),
            pl.BlockSpec((h, 1), lambda i: (0, 0)),
            pl.BlockSpec((h, o), lambda i: (0, 0)),
            pl.BlockSpec((o, 1), lambda i: (0, 0)),
        ],
        out_specs=pl.BlockSpec((rows_out, n), lambda i: (i, 0)),
        out_shape=jax.ShapeDtypeStruct((bt * o, n), jnp.float32),
    )(xp, adj, W1, b1[:, None], W2, b2[:, None])
    return outp.reshape(bt, o, n).transpose(0, 2, 1)


# speedup vs baseline: 1.2397x; 1.2397x over previous
import jax
import jax.numpy as jnp
from jax.experimental import pallas as pl
from jax.experimental.pallas import tpu as pltpu


def _gcn2_kernel(xp_ref, adj_ref, w1_ref, b1_ref, w2_ref, b2_ref, out_ref):
    adjv = adj_ref[...]
    n = adjv.shape[0]
    dis = jax.lax.rsqrt(1.0 + jnp.sum(adjv, axis=0, keepdims=True))
    r = jax.lax.broadcasted_iota(jnp.int32, (n, n), 0)
    c = jax.lax.broadcasted_iota(jnp.int32, (n, n), 1)
    ahat = (adjv + jnp.where(r == c, 1.0, 0.0)) * dis

    w1t = w1_ref[...].T
    w2t = w2_ref[...].T
    b1 = b1_ref[...]
    b2 = b2_ref[...]
    f1 = w1t.shape[1]
    f2 = w2t.shape[1]
    nb = xp_ref.shape[0] // f1

    def layer(hcur, wt, f, bias):
        agg = jnp.dot(hcur * dis, ahat, preferred_element_type=jnp.float32)
        pieces = [
            jnp.maximum(
                jnp.dot(wt, agg[i * f:(i + 1) * f, :],
                        preferred_element_type=jnp.float32) + bias, 0.0)
            for i in range(nb)
        ]
        return jnp.concatenate(pieces, axis=0)

    h1 = layer(xp_ref[...], w1t, f1, b1)
    out_ref[...] = layer(h1, w2t, f2, b2)


def kernel(x, adj, W1, b1, W2, b2):
    bt, n, f = x.shape
    o = W2.shape[1]
    h = W1.shape[1]
    xp = x.transpose(0, 2, 1).reshape(bt * f, n)
    grid = 2
    rows_in = bt * f // grid
    rows_out = bt * o // grid
    outp = pl.pallas_call(
        _gcn2_kernel,
        grid=(grid,),
        in_specs=[
            pl.BlockSpec((rows_in, n), lambda i: (i, 0)),
            pl.BlockSpec((n, n), lambda i: (0, 0)),
            pl.BlockSpec((f, h), lambda i: (0, 0)),
            pl.BlockSpec((h, 1), lambda i: (0, 0)),
            pl.BlockSpec((h, o), lambda i: (0, 0)),
            pl.BlockSpec((o, 1), lambda i: (0, 0)),
        ],
        out_specs=pl.BlockSpec((rows_out, n), lambda i: (i, 0)),
        out_shape=jax.ShapeDtypeStruct((bt * o, n), jnp.float32),
        compiler_params=pltpu.CompilerParams(
            dimension_semantics=("parallel",)),
    )(xp, adj, W1, b1[:, None], W2, b2[:, None])
    return outp.reshape(bt, o, n).transpose(0, 2, 1)


# speedup vs baseline: 1.2679x; 1.0227x over previous
import jax
import jax.numpy as jnp
from jax.experimental import pallas as pl


def _gcn2_kernel(xp_ref, adj_ref, w1_ref, b1_ref, w2_ref, b2_ref, out_ref):
    adjv = adj_ref[...]
    n = adjv.shape[0]
    dis = jax.lax.rsqrt(1.0 + jnp.sum(adjv, axis=0, keepdims=True))
    r = jax.lax.broadcasted_iota(jnp.int32, (n, n), 0)
    c = jax.lax.broadcasted_iota(jnp.int32, (n, n), 1)
    ahat = (adjv + jnp.where(r == c, 1.0, 0.0)) * dis

    w1t = w1_ref[...].T
    w2t = w2_ref[...].T
    b1 = b1_ref[...]
    b2 = b2_ref[...]
    f1 = w1t.shape[1]
    f2 = w2t.shape[1]
    nb = xp_ref.shape[0] // f1

    def layer(hcur, wt, f, bias):
        agg = jnp.dot(hcur * dis, ahat, preferred_element_type=jnp.float32)
        pieces = [
            jnp.maximum(
                jnp.dot(wt, agg[i * f:(i + 1) * f, :],
                        preferred_element_type=jnp.float32) + bias, 0.0)
            for i in range(nb)
        ]
        return jnp.concatenate(pieces, axis=0)

    h1 = layer(xp_ref[...], w1t, f1, b1)
    out_ref[...] = layer(h1, w2t, f2, b2)


def kernel(x, adj, W1, b1, W2, b2):
    bt, n, f = x.shape
    o = W2.shape[1]
    xp = x.transpose(0, 2, 1).reshape(bt * f, n)
    outp = pl.pallas_call(
        _gcn2_kernel,
        out_shape=jax.ShapeDtypeStruct((bt * o, n), jnp.float32),
    )(xp, adj, W1, b1[:, None], W2, b2[:, None])
    return outp.reshape(bt, o, n).transpose(0, 2, 1)
